# baseline (speedup 1.0000x reference)
import functools

import jax
import jax.numpy as jnp
import numpy as np
from jax.experimental import pallas as pl
from jax.experimental.pallas import tpu as pltpu

_K1, _K2 = 8, 8
_S1, _S2 = 4, 4
_N = 64
_H = _S1 * (_N - 1) + _K1
_W = _S2 * (_N - 1) + _K2

_E = np.zeros((_K2 * _N, _W), dtype=np.float32)
for _dj in range(_K2):
    for _pj in range(_N):
        _E[_dj * _N + _pj, _S2 * _pj + _dj] = 1.0


def _fold_kernel(x_ref, e_ref, o_ref):
    e = e_ref[...]

    def deinterleave(t8):
        bp = jnp.stack([t8[:, _N * p:_N * (p + 1)] for p in range(_N)],
                       axis=0)
        return jnp.concatenate([bp[:, j, :] for j in range(_K2)], axis=-1)

    for cc in range(2):
        slab = x_ref[0, cc].reshape(_K1 * _K2, _N * _N)
        rows = []
        for r in range(_S1):
            wt = deinterleave(slab[8 * r:8 * r + 8, :])
            wb = deinterleave(slab[8 * (r + _S1):8 * (r + _S1) + 8, :])
            zrow = jnp.zeros((1, _K2 * _N), dtype=wt.dtype)
            v = (jnp.concatenate([wt, zrow], axis=0)
                 + jnp.concatenate([zrow, wb], axis=0))
            rows.append(jnp.dot(v, e, preferred_element_type=jnp.float32))
        z = jnp.stack(rows, axis=1)
        o_ref[0, cc] = z.reshape(_H, _W)


@functools.partial(jax.jit, static_argnames=())
def kernel(x):
    b, c, k1, k2, l = x.shape
    n = _N
    e = jnp.asarray(_E)

    out = pl.pallas_call(
        _fold_kernel,
        grid=(b * c // 2,),
        in_specs=[
            pl.BlockSpec((1, 2, k1, k2, l), lambda i: (i // 16, i % 16, 0, 0, 0)),
            pl.BlockSpec((k2 * n, _W), lambda i: (0, 0)),
        ],
        out_specs=pl.BlockSpec((1, 2, _H, _W), lambda i: (i // 16, i % 16, 0, 0)),
        out_shape=jax.ShapeDtypeStruct((b, c, _H, _W), jnp.float32),
        compiler_params=pltpu.CompilerParams(
            dimension_semantics=(pltpu.GridDimensionSemantics.PARALLEL,),
            vmem_limit_bytes=50 * 1024 * 1024,
        ),
    )(x, e)
    return out


# speedup vs baseline: 1.0180x; 1.0180x over previous
import functools

import jax
import jax.numpy as jnp
import numpy as np
from jax.experimental import pallas as pl
from jax.experimental.pallas import tpu as pltpu

_K1, _K2 = 8, 8
_S1, _S2 = 4, 4
_N = 64
_H = _S1 * (_N - 1) + _K1
_W = _S2 * (_N - 1) + _K2

_E = np.zeros((_K2 * _N, _W), dtype=np.float32)
for _dj in range(_K2):
    for _pj in range(_N):
        _E[_dj * _N + _pj, _S2 * _pj + _dj] = 1.0


def _fold_kernel(x_ref, e_ref, o_ref):
    e = e_ref[...]

    def deinterleave(t8):
        bp = jnp.stack([t8[:, _N * p:_N * (p + 1)] for p in range(_N)],
                       axis=0)
        return jnp.concatenate([bp[:, j, :] for j in range(_K2)], axis=-1)

    for cc in range(4):
        slab = x_ref[0, cc].reshape(_K1 * _K2, _N * _N)
        rows = []
        for r in range(_S1):
            wt = deinterleave(slab[8 * r:8 * r + 8, :])
            wb = deinterleave(slab[8 * (r + _S1):8 * (r + _S1) + 8, :])
            zrow = jnp.zeros((1, _K2 * _N), dtype=wt.dtype)
            v = (jnp.concatenate([wt, zrow], axis=0)
                 + jnp.concatenate([zrow, wb], axis=0))
            rows.append(jnp.dot(v, e, preferred_element_type=jnp.float32))
        z = jnp.stack(rows, axis=1)
        o_ref[0, cc] = z.reshape(_H, _W)


@functools.partial(jax.jit, static_argnames=())
def kernel(x):
    b, c, k1, k2, l = x.shape
    n = _N
    e = jnp.asarray(_E)

    out = pl.pallas_call(
        _fold_kernel,
        grid=(b * c // 4,),
        in_specs=[
            pl.BlockSpec((1, 4, k1, k2, l), lambda i: (i // 8, i % 8, 0, 0, 0)),
            pl.BlockSpec((k2 * n, _W), lambda i: (0, 0)),
        ],
        out_specs=pl.BlockSpec((1, 4, _H, _W), lambda i: (i // 8, i % 8, 0, 0)),
        out_shape=jax.ShapeDtypeStruct((b, c, _H, _W), jnp.float32),
        compiler_params=pltpu.CompilerParams(
            dimension_semantics=(pltpu.GridDimensionSemantics.PARALLEL,),
            vmem_limit_bytes=50 * 1024 * 1024,
        ),
    )(x, e)
    return out


# speedup vs baseline: 1.0191x; 1.0011x over previous
import functools

import jax
import jax.numpy as jnp
import numpy as np
from jax.experimental import pallas as pl
from jax.experimental.pallas import tpu as pltpu

_K1, _K2 = 8, 8
_S1, _S2 = 4, 4
_N = 64
_H = _S1 * (_N - 1) + _K1
_W = _S2 * (_N - 1) + _K2

_E = np.zeros((_K2 * _N, _W), dtype=np.float32)
for _dj in range(_K2):
    for _pj in range(_N):
        _E[_dj * _N + _pj, _S2 * _pj + _dj] = 1.0


def _fold_kernel(x_ref, e_ref, o_ref):
    e = e_ref[...]

    def deinterleave(t8):
        bp = jnp.stack([t8[:, _N * p:_N * (p + 1)] for p in range(_N)],
                       axis=0)
        return jnp.concatenate([bp[:, j, :] for j in range(_K2)], axis=-1)

    for cc in range(8):
        slab = x_ref[0, cc].reshape(_K1 * _K2, _N * _N)
        rows = []
        for r in range(_S1):
            wt = deinterleave(slab[8 * r:8 * r + 8, :])
            wb = deinterleave(slab[8 * (r + _S1):8 * (r + _S1) + 8, :])
            zrow = jnp.zeros((1, _K2 * _N), dtype=wt.dtype)
            v = (jnp.concatenate([wt, zrow], axis=0)
                 + jnp.concatenate([zrow, wb], axis=0))
            rows.append(jnp.dot(v, e, preferred_element_type=jnp.float32))
        z = jnp.stack(rows, axis=1)
        o_ref[0, cc] = z.reshape(_H, _W)


@functools.partial(jax.jit, static_argnames=())
def kernel(x):
    b, c, k1, k2, l = x.shape
    n = _N
    e = jnp.asarray(_E)

    out = pl.pallas_call(
        _fold_kernel,
        grid=(b * c // 8,),
        in_specs=[
            pl.BlockSpec((1, 8, k1, k2, l), lambda i: (i // 4, i % 4, 0, 0, 0)),
            pl.BlockSpec((k2 * n, _W), lambda i: (0, 0)),
        ],
        out_specs=pl.BlockSpec((1, 8, _H, _W), lambda i: (i // 4, i % 4, 0, 0)),
        out_shape=jax.ShapeDtypeStruct((b, c, _H, _W), jnp.float32),
        compiler_params=pltpu.CompilerParams(
            dimension_semantics=(pltpu.GridDimensionSemantics.PARALLEL,),
            vmem_limit_bytes=50 * 1024 * 1024,
        ),
    )(x, e)
    return out
